# Initial kernel scaffold; baseline (speedup 1.0000x reference)
import jax
import jax.numpy as jnp
from jax.experimental import pallas as pl


def kernel(x, lateral_weights, activation, spikes, threshold, freq_act):
    raise NotImplementedError("write your pallas kernel here")


# speedup vs baseline: 7.5684x; 7.5684x over previous
import functools

import jax
import jax.numpy as jnp
from jax import lax
from jax.experimental import pallas as pl
from jax.experimental.pallas import tpu as pltpu
from jax.experimental.pallas import tpu_sc as plsc

_SHAPE = (96, 96)
_N = _SHAPE[0] * _SHAPE[1]
_BETA = 0.9
_L = 16
_NC = 2
_NS = 16
_NW = _NC * _NS
_TW = 128
_NT = _N // _TW
_TPW = (_NT + _NW - 1) // _NW
_ROWS = 16


def _sc_body(spk_hbm, w_hbm, x_hbm, a_hbm, t_hbm, out_hbm,
             spk_v, idx_v, rows_v, acc_v, x_v, a_v, t_v, o_v, sem):
    cid = lax.axis_index("c")
    sid = lax.axis_index("s")
    wid = sid * _NC + cid

    pltpu.sync_copy(spk_hbm, spk_v)

    lanes = lax.iota(jnp.int32, _L)

    def scan_body(i, kvec):
        v = spk_v[pl.ds(i * _L, _L)]
        m = v != 0
        pos = kvec + lax.cumsum(v) - 1
        plsc.store_scatter(idx_v, [pos], lanes + i * _L, mask=m)
        return kvec + plsc.all_reduce_population_count(m)

    kvec = lax.fori_loop(0, _N // _L, scan_body,
                         jnp.zeros((_L,), jnp.int32))
    k_total = jnp.max(kvec)

    plsc.store_scatter(idx_v, [k_total + lanes], jnp.zeros((_L,), jnp.int32))

    nfull = k_total // _ROWS
    tail = k_total - nfull * _ROWS
    zf = jnp.zeros((_L,), jnp.float32)

    for t in range(_TPW):
        tile = wid + _NW * t

        @pl.when(tile < _NT)
        def _():
            c0 = pl.multiple_of(tile * _TW, _TW)

            for i in range(_TW // _L):
                acc_v[pl.ds(i * _L, _L)] = zf

            def batch_body(b, carry):
                src = w_hbm.at[idx_v.at[pl.ds(b * _ROWS, _ROWS)],
                               pl.ds(c0, _TW)]
                pltpu.async_copy(src, rows_v, sem).wait()
                for r in range(_ROWS):
                    for ci in range(_TW // _L):
                        d = pl.ds(ci * _L, _L)
                        acc_v[d] = acc_v[d] + rows_v[r, d]
                return carry

            lax.fori_loop(0, nfull, batch_body, 0)

            @pl.when(tail > 0)
            def _():
                src = w_hbm.at[idx_v.at[pl.ds(nfull * _ROWS, _ROWS)],
                               pl.ds(c0, _TW)]
                pltpu.async_copy(src, rows_v, sem).wait()
                for r in range(_ROWS - 1):
                    @pl.when(r < tail)
                    def _():
                        for ci in range(_TW // _L):
                            d = pl.ds(ci * _L, _L)
                            acc_v[d] = acc_v[d] + rows_v[r, d]

            pltpu.sync_copy(x_hbm.at[pl.ds(c0, _TW)], x_v)
            pltpu.sync_copy(a_hbm.at[pl.ds(c0, _TW)], a_v)
            pltpu.sync_copy(t_hbm.at[pl.ds(c0, _TW)], t_v)
            for ci in range(_TW // _L):
                d = pl.ds(ci * _L, _L)
                act = _BETA * a_v[d] + x_v[d] + acc_v[d]
                o_v[d] = jnp.maximum(jnp.sign(act - t_v[d]), 0.0)
            pltpu.sync_copy(o_v, out_hbm.at[pl.ds(c0, _TW)])


@functools.cache
def _sc_step():
    return pl.kernel(
        _sc_body,
        out_type=jax.ShapeDtypeStruct((_N,), jnp.float32),
        mesh=plsc.VectorSubcoreMesh(core_axis_name="c", subcore_axis_name="s",
                                    num_cores=_NC, num_subcores=_NS),
        compiler_params=pltpu.CompilerParams(needs_layout_passes=False),
        scratch_types=[
            pltpu.VMEM((_N,), jnp.int32),
            pltpu.VMEM((_N + _L,), jnp.int32),
            pltpu.VMEM((_ROWS, _TW), jnp.float32),
            pltpu.VMEM((_TW,), jnp.float32),
            pltpu.VMEM((_TW,), jnp.float32),
            pltpu.VMEM((_TW,), jnp.float32),
            pltpu.VMEM((_TW,), jnp.float32),
            pltpu.VMEM((_TW,), jnp.float32),
            pltpu.SemaphoreType.DMA,
        ],
    )


def kernel(x, lateral_weights, activation, spikes, threshold, freq_act):
    del freq_act
    spk = spikes.reshape(-1).astype(jnp.int32)
    out = _sc_step()(spk, lateral_weights, x.reshape(-1),
                     activation.reshape(-1), threshold.reshape(-1))
    return out.reshape(_SHAPE).astype(bool)


# speedup vs baseline: 10.5013x; 1.3875x over previous
import functools

import jax
import jax.numpy as jnp
from jax import lax
from jax.experimental import pallas as pl
from jax.experimental.pallas import tpu as pltpu
from jax.experimental.pallas import tpu_sc as plsc

_SHAPE = (96, 96)
_N = _SHAPE[0] * _SHAPE[1]
_NWORDS = _N // 4
_BETA = 0.9
_L = 16
_NC = 2
_NS = 16
_NW = _NC * _NS
_TW = 128
_NT = _N // _TW
_TPW = (_NT + _NW - 1) // _NW
_ROWS = 16


def _sc_body(spk_hbm, w_hbm, x_hbm, a_hbm, t_hbm, out_hbm,
             spk_v, wrd_v, idx_v, rows_v, acc_v, x_v, a_v, t_v, o_v, sem):
    cid = lax.axis_index("c")
    sid = lax.axis_index("s")
    wid = sid * _NC + cid

    pltpu.sync_copy(spk_hbm, spk_v)

    lanes = lax.iota(jnp.int32, _L)
    zi = jnp.zeros((_L,), jnp.int32)

    def or_body(i, o):
        return o | spk_v[pl.ds(i * _L, _L)]

    orv = lax.fori_loop(0, _NWORDS // _L, or_body, zi)
    any_spike = jnp.max(orv)

    def compact():
        def l1(i, nwvec):
            w = spk_v[pl.ds(i * _L, _L)]
            m = w != 0
            pos = nwvec + lax.cumsum(m.astype(jnp.int32)) - 1
            plsc.store_scatter(wrd_v, [pos], lanes + i * _L, mask=m)
            return nwvec + plsc.all_reduce_population_count(m)

        nwvec = lax.fori_loop(0, _NWORDS // _L, l1, zi)
        nw = jnp.max(nwvec)
        plsc.store_scatter(wrd_v, [nw + lanes], zi)

        def l2(j, kvec):
            widx = wrd_v[pl.ds(j * _L, _L)]
            w = plsc.load_gather(spk_v, [widx])
            valid = (j * _L + lanes) < nw
            k2 = kvec
            for b in range(4):
                vb = (w >> (8 * b)) & 0xFF
                mb = (vb != 0) & valid
                pos = k2 + lax.cumsum(mb.astype(jnp.int32)) - 1
                plsc.store_scatter(idx_v, [pos], widx * 4 + b, mask=mb)
                k2 = k2 + plsc.all_reduce_population_count(mb)
            return k2

        nbw = (nw + _L - 1) // _L
        kvec = lax.fori_loop(0, nbw, l2, zi)
        k = jnp.max(kvec)
        plsc.store_scatter(idx_v, [k + lanes], zi)
        return k

    k_total = lax.cond(any_spike != 0, compact, lambda: jnp.int32(0))

    nfull = k_total // _ROWS
    tail = k_total - nfull * _ROWS
    zf = jnp.zeros((_L,), jnp.float32)

    for t in range(_TPW):
        tile = wid + _NW * t

        @pl.when(tile < _NT)
        def _():
            c0 = pl.multiple_of(tile * _TW, _TW)

            cx = pltpu.async_copy(x_hbm.at[pl.ds(c0, _TW)], x_v, sem)
            ca = pltpu.async_copy(a_hbm.at[pl.ds(c0, _TW)], a_v, sem)
            ct = pltpu.async_copy(t_hbm.at[pl.ds(c0, _TW)], t_v, sem)

            for i in range(_TW // _L):
                acc_v[pl.ds(i * _L, _L)] = zf

            def batch_body(b, carry):
                src = w_hbm.at[idx_v.at[pl.ds(b * _ROWS, _ROWS)],
                               pl.ds(c0, _TW)]
                pltpu.async_copy(src, rows_v, sem).wait()
                for r in range(_ROWS):
                    for ci in range(_TW // _L):
                        d = pl.ds(ci * _L, _L)
                        acc_v[d] = acc_v[d] + rows_v[r, d]
                return carry

            lax.fori_loop(0, nfull, batch_body, 0)

            @pl.when(tail > 0)
            def _():
                src = w_hbm.at[idx_v.at[pl.ds(nfull * _ROWS, _ROWS)],
                               pl.ds(c0, _TW)]
                pltpu.async_copy(src, rows_v, sem).wait()
                for r in range(_ROWS - 1):
                    @pl.when(r < tail)
                    def _():
                        for ci in range(_TW // _L):
                            d = pl.ds(ci * _L, _L)
                            acc_v[d] = acc_v[d] + rows_v[r, d]

            cx.wait()
            ca.wait()
            ct.wait()
            for ci in range(_TW // _L):
                d = pl.ds(ci * _L, _L)
                act = _BETA * a_v[d] + x_v[d] + acc_v[d]
                o_v[d] = (act > t_v[d]).astype(jnp.float32)
            pltpu.sync_copy(o_v, out_hbm.at[pl.ds(c0, _TW)])


@functools.cache
def _sc_step():
    return pl.kernel(
        _sc_body,
        out_type=jax.ShapeDtypeStruct((_N,), jnp.float32),
        mesh=plsc.VectorSubcoreMesh(core_axis_name="c", subcore_axis_name="s",
                                    num_cores=_NC, num_subcores=_NS),
        compiler_params=pltpu.CompilerParams(needs_layout_passes=False),
        scratch_types=[
            pltpu.VMEM((_NWORDS,), jnp.int32),
            pltpu.VMEM((_NWORDS + _L,), jnp.int32),
            pltpu.VMEM((_N + _L,), jnp.int32),
            pltpu.VMEM((_ROWS, _TW), jnp.float32),
            pltpu.VMEM((_TW,), jnp.float32),
            pltpu.VMEM((_TW,), jnp.float32),
            pltpu.VMEM((_TW,), jnp.float32),
            pltpu.VMEM((_TW,), jnp.float32),
            pltpu.VMEM((_TW,), jnp.float32),
            pltpu.SemaphoreType.DMA,
        ],
    )


def kernel(x, lateral_weights, activation, spikes, threshold, freq_act):
    del freq_act
    s = spikes.reshape(_NWORDS, 4).astype(jnp.int32)
    spk = s[:, 0] | (s[:, 1] << 8) | (s[:, 2] << 16) | (s[:, 3] << 24)
    out = _sc_step()(spk, lateral_weights, x.reshape(-1),
                     activation.reshape(-1), threshold.reshape(-1))
    return out.reshape(_SHAPE).astype(bool)
